# baseline (speedup 1.0000x reference)
import functools

import jax
import jax.numpy as jnp
from jax import lax
from jax.experimental import pallas as pl
from jax.experimental.pallas import tpu as pltpu
from jax.experimental.pallas import tpu_sc as plsc

D_MODEL = 64
SCALE = 8.0

NC = 2
NS = 16
NW = NC * NS


def _tc_transpose_lut(lut_t):
    V = lut_t.shape[1]
    BLK = 512
    grid = (V + BLK - 1) // BLK

    def body(in_ref, out_ref):
        out_ref[...] = in_ref[...].T

    return pl.pallas_call(
        body,
        grid=(grid,),
        in_specs=[pl.BlockSpec((D_MODEL, BLK), lambda i: (0, i))],
        out_specs=pl.BlockSpec((BLK, D_MODEL), lambda i: (i, 0)),
        out_shape=jax.ShapeDtypeStruct((V, D_MODEL), jnp.float32),
        compiler_params=pltpu.CompilerParams(
            dimension_semantics=("parallel",),
        ),
    )(lut_t)


def _tc_transpose_scale_out(out2):
    S, B, _ = out2.shape
    BLK = 512

    def body(in_ref, out_ref):
        out_ref[...] = jnp.swapaxes(in_ref[...], 1, 2) * SCALE

    return pl.pallas_call(
        body,
        grid=(S, B // BLK),
        in_specs=[pl.BlockSpec((1, BLK, D_MODEL), lambda s, j: (s, j, 0))],
        out_specs=pl.BlockSpec((1, D_MODEL, BLK), lambda s, j: (s, 0, j)),
        out_shape=jax.ShapeDtypeStruct((S, D_MODEL, B), jnp.float32),
        compiler_params=pltpu.CompilerParams(
            dimension_semantics=("parallel", "parallel"),
        ),
    )(out2)


def _sc_gather(x_t, lut_rows):
    S, B = x_t.shape
    b_per_w = B // NW
    mesh = plsc.VectorSubcoreMesh(core_axis_name="c", subcore_axis_name="s")

    @functools.partial(
        pl.kernel,
        mesh=mesh,
        out_type=jax.ShapeDtypeStruct((S * B, D_MODEL), jnp.float32),
        compiler_params=pltpu.CompilerParams(use_tc_tiling_on_sc=False),
        scratch_types=[
            pltpu.VMEM((b_per_w,), jnp.int32),
            pltpu.VMEM((b_per_w, D_MODEL), jnp.float32),
            pltpu.SemaphoreType.DMA,
        ],
    )
    def k(lut_hbm, idx_hbm, out_hbm, idx_v, rows_v, sem):
        wid = lax.axis_index("s") * NC + lax.axis_index("c")
        wbase = wid * b_per_w

        @pl.loop(0, S)
        def _(s):
            pltpu.sync_copy(idx_hbm.at[s, pl.ds(wbase, b_per_w)], idx_v)
            pltpu.async_copy(lut_hbm.at[idx_v], rows_v, sem).wait()
            pltpu.sync_copy(rows_v, out_hbm.at[pl.ds(s * B + wbase, b_per_w)])

    return k(lut_rows, x_t)


def kernel(x, lut):
    S_B = x.shape
    x_t = jnp.swapaxes(x.astype(jnp.int32), 0, 1)
    lut_t = jnp.swapaxes(lut, 0, 1)
    lut_rows = _tc_transpose_lut(lut_t)
    out2 = _sc_gather(x_t, lut_rows)
    out3 = out2.reshape(x_t.shape[0], x_t.shape[1], D_MODEL)
    out_t = _tc_transpose_scale_out(out3)
    return jnp.transpose(out_t, (2, 0, 1))


# speedup vs baseline: 4.6179x; 4.6179x over previous
import functools

import jax
import jax.numpy as jnp
from jax import lax
from jax.experimental import pallas as pl
from jax.experimental.pallas import tpu as pltpu
from jax.experimental.pallas import tpu_sc as plsc

D_MODEL = 64
SCALE = 8.0

NC = 2
NS = 16
NW = NC * NS

LUT_BLK = 8192
LUT_HALF = LUT_BLK // 2
CHUNK = 1024


def _tc_pack_lut(lut_t, vp):
    V = lut_t.shape[1]
    grid = (V + LUT_BLK - 1) // LUT_BLK

    def body(in_ref, out_ref):
        t = in_ref[...].T
        out_ref[...] = jnp.concatenate([t[:LUT_HALF], t[LUT_HALF:]], axis=1)

    return pl.pallas_call(
        body,
        grid=(grid,),
        in_specs=[pl.BlockSpec((D_MODEL, LUT_BLK), lambda i: (0, i))],
        out_specs=pl.BlockSpec((LUT_HALF, 128), lambda i: (i, 0)),
        out_shape=jax.ShapeDtypeStruct((vp // 2, 128), jnp.float32),
        compiler_params=pltpu.CompilerParams(
            dimension_semantics=("parallel",),
        ),
    )(lut_t)


def _tc_unpack_scale_out(out_pairs, S, B):
    half = B // 2

    def body(in_ref, out_ref):
        x = in_ref[...]
        lo = x[:, :D_MODEL].T
        hi = x[:, D_MODEL:].T
        out_ref[...] = (jnp.concatenate([lo, hi], axis=1) * SCALE).reshape(
            1, D_MODEL, B
        )

    return pl.pallas_call(
        body,
        grid=(S,),
        in_specs=[pl.BlockSpec((half, 128), lambda s: (s, 0))],
        out_specs=pl.BlockSpec((1, D_MODEL, B), lambda s: (s, 0, 0)),
        out_shape=jax.ShapeDtypeStruct((S, D_MODEL, B), jnp.float32),
        compiler_params=pltpu.CompilerParams(
            dimension_semantics=("parallel",),
        ),
    )(out_pairs)


def _sc_gather(x_flat, lut_rows, S, B):
    n_idx = S * B
    n_chunks = n_idx // (NW * CHUNK)
    half = B // 2
    mesh = plsc.VectorSubcoreMesh(core_axis_name="c", subcore_axis_name="s")

    @functools.partial(
        pl.kernel,
        mesh=mesh,
        out_type=jax.ShapeDtypeStruct((n_idx // 2, 128), jnp.float32),
        compiler_params=pltpu.CompilerParams(use_tc_tiling_on_sc=False),
        scratch_types=[
            pltpu.VMEM((CHUNK,), jnp.int32),
            pltpu.VMEM((CHUNK, D_MODEL), jnp.float32),
            pltpu.SemaphoreType.DMA,
        ],
    )
    def k(lut_hbm, idx_hbm, out_hbm, idx_v, rows_v, sem):
        wid = lax.axis_index("s") * NC + lax.axis_index("c")

        @pl.loop(0, n_chunks)
        def _(g):
            kc = wid * n_chunks + g
            base = kc * CHUNK
            pltpu.sync_copy(idx_hbm.at[pl.ds(base, CHUNK)], idx_v)

            @pl.loop(0, CHUNK, step=16)
            def _(o):
                v = idx_v.at[pl.ds(o, 16)][...]
                u = jnp.bitwise_and(v, LUT_BLK - 1)
                j = (
                    (v - u)
                    + jnp.left_shift(jnp.bitwise_and(u, LUT_HALF - 1), 1)
                    + jnp.right_shift(u, 12)
                )
                idx_v.at[pl.ds(o, 16)][...] = j

            pltpu.async_copy(lut_hbm.at[idx_v], rows_v, sem).wait()

            s = kc // 4
            q = kc - s * 4
            h = q // 2
            pairbase = s * half + (q - h * 2) * CHUNK
            pltpu.sync_copy(
                rows_v,
                out_hbm.at[pl.ds(pairbase, CHUNK), pl.ds(h * D_MODEL, D_MODEL)],
            )

    return k(lut_rows, x_flat)


def kernel(x, lut):
    B, S = x.shape
    V = lut.shape[0]
    n_blocks = (V + LUT_BLK - 1) // LUT_BLK
    vp = n_blocks * LUT_BLK
    x_t = jnp.swapaxes(x.astype(jnp.int32), 0, 1)
    x_flat = x_t.reshape(-1)
    lut_t = jnp.swapaxes(lut, 0, 1)
    lut_pairs = _tc_pack_lut(lut_t, vp)
    lut_rows = lut_pairs.reshape(vp, D_MODEL)
    out_pairs = _sc_gather(x_flat, lut_rows, S, B)
    out_t = _tc_unpack_scale_out(out_pairs, S, B)
    return jnp.transpose(out_t, (2, 0, 1))
